# baseline (speedup 1.0000x reference)
import functools

import jax
import jax.numpy as jnp
from jax import lax
from jax.experimental import pallas as pl
from jax.experimental.pallas import tpu as pltpu
from jax.experimental.pallas import tpu_sc as plsc

D = 128
W = 5
S = 5
M = 65536
K = 8
BLK = 16384
G = M // BLK
NCOLS = M + 128


def _rsqrt_precise(x):
    y = lax.rsqrt(x)
    y = y * (1.5 - 0.5 * x * y * y)
    return y * (1.5 - 0.5 * x * y * y)


def _tc_body(sup_ref, mem_ref, ps_ref, w_ref, idx_ref, den_ref,
             tv_ref, ti_ref):
    i = pl.program_id(0)

    sup = sup_ref[...]
    n2s = jnp.sum(sup * sup, axis=1, keepdims=True)
    shat = sup * _rsqrt_precise(jnp.maximum(n2s, 1e-24))
    q = (shat[0:5] + shat[5:10] + shat[10:15] + shat[15:20] + shat[20:25]) * 0.2

    mblk = mem_ref[...]
    n2m = lax.dot_general(
        jnp.ones((1, D), jnp.float32), mblk * mblk,
        (((1,), (1,)), ((), ())),
        preferred_element_type=jnp.float32,
    )
    rn = _rsqrt_precise(jnp.maximum(n2m, 1e-24))
    sc = lax.dot_general(
        q, mblk, (((1,), (1,)), ((), ())),
        preferred_element_type=jnp.float32,
    ) * rn

    giota = lax.broadcasted_iota(jnp.int32, (W, BLK), 1) + i * BLK
    bvals, bidxs = [], []
    for _ in range(K):
        m = jnp.max(sc, axis=1, keepdims=True)
        am = jnp.min(jnp.where(sc >= m, giota, 2**30),
                     axis=1, keepdims=True)
        bvals.append(m)
        bidxs.append(am)
        sc = jnp.where(giota == am, -1e30, sc)
    bv = jnp.concatenate(bvals, axis=1)
    bi = jnp.concatenate(bidxs, axis=1)

    def _pad_store(v8, i8):
        tv_ref[...] = jnp.concatenate(
            [v8, jnp.full((W, 128 - K), -1e30, jnp.float32)], axis=1)
        ti_ref[...] = jnp.concatenate(
            [i8, jnp.zeros((W, 128 - K), jnp.int32)], axis=1)

    def _top8_of(cand_v, cand_i, n):
        liota = lax.broadcasted_iota(jnp.int32, (W, n), 1)
        vals, idxs = [], []
        for _ in range(K):
            m = jnp.max(cand_v, axis=1, keepdims=True)
            am = jnp.min(jnp.where(cand_v >= m, liota, 2**30),
                         axis=1, keepdims=True)
            gi = jnp.sum(jnp.where(liota == am, cand_i, 0),
                         axis=1, keepdims=True)
            vals.append(m)
            idxs.append(gi)
            cand_v = jnp.where(liota == am, -1e30, cand_v)
        return (jnp.concatenate(vals, axis=1),
                jnp.concatenate(idxs, axis=1))

    @pl.when(i == 0)
    def _init():
        _pad_store(bv, bi)

    @pl.when(i > 0)
    def _merge():
        rv = tv_ref[...][:, 0:K]
        ri = ti_ref[...][:, 0:K]
        mv, mi = _top8_of(jnp.concatenate([rv, bv], axis=1),
                          jnp.concatenate([ri, bi], axis=1), 2 * K)
        _pad_store(mv, mi)

    @pl.when(i == G - 1)
    def _finalize():
        cols = [
            jnp.sum(q * shat[j * W:(j + 1) * W], axis=1, keepdims=True)
            for j in range(S)
        ]
        supsc = jnp.concatenate(cols, axis=1)
        sup_gidx = lax.broadcasted_iota(jnp.int32, (W, S), 1) + M
        rv = tv_ref[...][:, 0:K]
        ri = ti_ref[...][:, 0:K]
        pad = jnp.full((W, 3), -1e30, jnp.float32)
        padi = jnp.zeros((W, 3), jnp.int32)
        V, I = _top8_of(jnp.concatenate([rv, supsc, pad], axis=1),
                        jnp.concatenate([ri, sup_gidx, padi], axis=1), 2 * K)

        is_mem = I < M
        w8 = jnp.where(is_mem, V, 0.0)
        i8 = jnp.where(is_mem, I, 0)
        zf = jnp.zeros((W, K), jnp.float32)
        zi = jnp.zeros((W, K), jnp.int32)
        w_ref[...] = jnp.concatenate([w8, zf], axis=1)
        idx_ref[...] = jnp.concatenate([i8, zi], axis=1)
        den = jnp.sum(V, axis=1, keepdims=True)
        den_ref[...] = jnp.broadcast_to(den, (W, 16))

        ps = jnp.zeros((W, D), jnp.float32)
        for j in range(S):
            swj = jnp.sum(jnp.where(I == M + j, V, 0.0), axis=1, keepdims=True)
            ps = ps + swj * sup[j * W:(j + 1) * W]
        ps_ref[...] = ps


def _scores_stage(sup, memory):
    return pl.pallas_call(
        _tc_body,
        grid=(G,),
        in_specs=[
            pl.BlockSpec((S * W, D), lambda i: (0, 0)),
            pl.BlockSpec((BLK, D), lambda i: (i, 0)),
        ],
        out_specs=[
            pl.BlockSpec((W, D), lambda i: (0, 0)),
            pl.BlockSpec((W, 16), lambda i: (0, 0)),
            pl.BlockSpec((W, 16), lambda i: (0, 0)),
            pl.BlockSpec((W, 16), lambda i: (0, 0)),
        ],
        out_shape=[
            jax.ShapeDtypeStruct((W, D), jnp.float32),
            jax.ShapeDtypeStruct((W, 16), jnp.float32),
            jax.ShapeDtypeStruct((W, 16), jnp.int32),
            jax.ShapeDtypeStruct((W, 16), jnp.float32),
        ],
        scratch_shapes=[
            pltpu.VMEM((W, 128), jnp.float32),
            pltpu.VMEM((W, 128), jnp.int32),
        ],
    )(sup, memory)


@functools.cache
def _build_sc_combine():
    @functools.partial(
        pl.kernel,
        mesh=plsc.VectorSubcoreMesh(core_axis_name="c", subcore_axis_name="s"),
        out_type=jax.ShapeDtypeStruct((W * D,), jnp.float32),
        scratch_types=[
            pltpu.VMEM((16,), jnp.int32),
            pltpu.VMEM((16, D), jnp.float32),
            pltpu.VMEM((16,), jnp.float32),
            pltpu.VMEM((16,), jnp.float32),
            pltpu.VMEM((D,), jnp.float32),
            pltpu.VMEM((D,), jnp.float32),
            pltpu.SemaphoreType.DMA,
        ],
    )
    def _sc_combine(mem_hbm, idx_hbm, w_hbm, den_hbm, ps_hbm, out_hbm,
                    idx_v, rows_v, w_v, den_v, ps_v, out_v, sem):
        _sc_body(mem_hbm, idx_hbm, w_hbm, den_hbm, ps_hbm, out_hbm,
                 idx_v, rows_v, w_v, den_v, ps_v, out_v, sem)

    return _sc_combine


def _sc_body(mem_hbm, idx_hbm, w_hbm, den_hbm, ps_hbm, out_hbm,
             idx_v, rows_v, w_v, den_v, ps_v, out_v, sem):
    wid = lax.axis_index("s") * 2 + lax.axis_index("c")

    @pl.when(wid < W)
    def _():
        pltpu.sync_copy(idx_hbm.at[pl.ds(wid * 16, 16)], idx_v)
        pltpu.sync_copy(w_hbm.at[pl.ds(wid * 16, 16)], w_v)
        pltpu.sync_copy(den_hbm.at[pl.ds(wid * 16, 16)], den_v)
        pltpu.sync_copy(ps_hbm.at[pl.ds(wid * D, D)], ps_v)
        pltpu.async_copy(mem_hbm.at[idx_v], rows_v, sem).wait()

        wvec = w_v[...]
        dvec = den_v[...]
        wks = [
            wvec.at[jnp.full((16,), k, jnp.int32)].get(mode="promise_in_bounds")
            for k in range(K)
        ]
        for c in range(D // 16):
            acc = ps_v[pl.ds(c * 16, 16)]
            for k in range(K):
                acc = acc + wks[k] * rows_v[k, pl.ds(c * 16, 16)]
            out_v[pl.ds(c * 16, 16)] = acc / dvec
        pltpu.sync_copy(out_v, out_hbm.at[pl.ds(wid * D, D)])


def kernel(support, memory_encoded):
    sup = support.reshape(S * W, D)
    ps, w16, i16, den16 = _scores_stage(sup, memory_encoded)
    proto = _build_sc_combine()(
        memory_encoded,
        i16.reshape(-1),
        w16.reshape(-1),
        den16.reshape(-1),
        ps.reshape(-1),
    )
    return proto.reshape(1, W, D)


# speedup vs baseline: 1.0909x; 1.0909x over previous
import functools

import jax
import jax.numpy as jnp
from jax import lax
from jax.experimental import pallas as pl
from jax.experimental.pallas import tpu as pltpu
from jax.experimental.pallas import tpu_sc as plsc

D = 128
W = 5
S = 5
M = 65536
K = 8
BLK = 65536
G = M // BLK
NCOLS = M + 128


def _rsqrt_precise(x):
    y = lax.rsqrt(x)
    y = y * (1.5 - 0.5 * x * y * y)
    return y * (1.5 - 0.5 * x * y * y)


def _tc_body(sup_ref, mem_ref, ps_ref, w_ref, idx_ref, den_ref,
             tv_ref, ti_ref):
    i = pl.program_id(0)

    sup = sup_ref[...]
    n2s = jnp.sum(sup * sup, axis=1, keepdims=True)
    shat = sup * _rsqrt_precise(jnp.maximum(n2s, 1e-24))
    q = (shat[0:5] + shat[5:10] + shat[10:15] + shat[15:20] + shat[20:25]) * 0.2

    mblk = mem_ref[...]
    n2m = lax.dot_general(
        jnp.ones((1, D), jnp.float32), mblk * mblk,
        (((1,), (1,)), ((), ())),
        preferred_element_type=jnp.float32,
    )
    rn = _rsqrt_precise(jnp.maximum(n2m, 1e-24))
    sc = lax.dot_general(
        q, mblk, (((1,), (1,)), ((), ())),
        preferred_element_type=jnp.float32,
    ) * rn

    giota = lax.broadcasted_iota(jnp.int32, (W, BLK), 1) + i * BLK
    bvals, bidxs = [], []
    for _ in range(K):
        m = jnp.max(sc, axis=1, keepdims=True)
        am = jnp.min(jnp.where(sc >= m, giota, 2**30),
                     axis=1, keepdims=True)
        bvals.append(m)
        bidxs.append(am)
        sc = jnp.where(giota == am, -1e30, sc)
    bv = jnp.concatenate(bvals, axis=1)
    bi = jnp.concatenate(bidxs, axis=1)

    def _pad_store(v8, i8):
        tv_ref[...] = jnp.concatenate(
            [v8, jnp.full((W, 128 - K), -1e30, jnp.float32)], axis=1)
        ti_ref[...] = jnp.concatenate(
            [i8, jnp.zeros((W, 128 - K), jnp.int32)], axis=1)

    def _top8_of(cand_v, cand_i, n):
        liota = lax.broadcasted_iota(jnp.int32, (W, n), 1)
        vals, idxs = [], []
        for _ in range(K):
            m = jnp.max(cand_v, axis=1, keepdims=True)
            am = jnp.min(jnp.where(cand_v >= m, liota, 2**30),
                         axis=1, keepdims=True)
            gi = jnp.sum(jnp.where(liota == am, cand_i, 0),
                         axis=1, keepdims=True)
            vals.append(m)
            idxs.append(gi)
            cand_v = jnp.where(liota == am, -1e30, cand_v)
        return (jnp.concatenate(vals, axis=1),
                jnp.concatenate(idxs, axis=1))

    @pl.when(i == 0)
    def _init():
        _pad_store(bv, bi)

    @pl.when(i > 0)
    def _merge():
        rv = tv_ref[...][:, 0:K]
        ri = ti_ref[...][:, 0:K]
        mv, mi = _top8_of(jnp.concatenate([rv, bv], axis=1),
                          jnp.concatenate([ri, bi], axis=1), 2 * K)
        _pad_store(mv, mi)

    @pl.when(i == G - 1)
    def _finalize():
        cols = [
            jnp.sum(q * shat[j * W:(j + 1) * W], axis=1, keepdims=True)
            for j in range(S)
        ]
        supsc = jnp.concatenate(cols, axis=1)
        sup_gidx = lax.broadcasted_iota(jnp.int32, (W, S), 1) + M
        rv = tv_ref[...][:, 0:K]
        ri = ti_ref[...][:, 0:K]
        pad = jnp.full((W, 3), -1e30, jnp.float32)
        padi = jnp.zeros((W, 3), jnp.int32)
        V, I = _top8_of(jnp.concatenate([rv, supsc, pad], axis=1),
                        jnp.concatenate([ri, sup_gidx, padi], axis=1), 2 * K)

        is_mem = I < M
        w8 = jnp.where(is_mem, V, 0.0)
        i8 = jnp.where(is_mem, I, 0)
        zf = jnp.zeros((W, K), jnp.float32)
        zi = jnp.zeros((W, K), jnp.int32)
        w_ref[...] = jnp.concatenate([w8, zf], axis=1)
        idx_ref[...] = jnp.concatenate([i8, zi], axis=1)
        den = jnp.sum(V, axis=1, keepdims=True)
        den_ref[...] = jnp.broadcast_to(den, (W, 16))

        ps = jnp.zeros((W, D), jnp.float32)
        for j in range(S):
            swj = jnp.sum(jnp.where(I == M + j, V, 0.0), axis=1, keepdims=True)
            ps = ps + swj * sup[j * W:(j + 1) * W]
        ps_ref[...] = ps


def _scores_stage(sup, memory):
    return pl.pallas_call(
        _tc_body,
        grid=(G,),
        in_specs=[
            pl.BlockSpec((S * W, D), lambda i: (0, 0)),
            pl.BlockSpec((BLK, D), lambda i: (i, 0)),
        ],
        out_specs=[
            pl.BlockSpec((W, D), lambda i: (0, 0)),
            pl.BlockSpec((W, 16), lambda i: (0, 0)),
            pl.BlockSpec((W, 16), lambda i: (0, 0)),
            pl.BlockSpec((W, 16), lambda i: (0, 0)),
        ],
        out_shape=[
            jax.ShapeDtypeStruct((W, D), jnp.float32),
            jax.ShapeDtypeStruct((W, 16), jnp.float32),
            jax.ShapeDtypeStruct((W, 16), jnp.int32),
            jax.ShapeDtypeStruct((W, 16), jnp.float32),
        ],
        scratch_shapes=[
            pltpu.VMEM((W, 128), jnp.float32),
            pltpu.VMEM((W, 128), jnp.int32),
        ],
    )(sup, memory)


@functools.cache
def _build_sc_combine():
    @functools.partial(
        pl.kernel,
        mesh=plsc.VectorSubcoreMesh(core_axis_name="c", subcore_axis_name="s"),
        out_type=jax.ShapeDtypeStruct((W * D,), jnp.float32),
        scratch_types=[
            pltpu.VMEM((16,), jnp.int32),
            pltpu.VMEM((16, D), jnp.float32),
            pltpu.VMEM((16,), jnp.float32),
            pltpu.VMEM((16,), jnp.float32),
            pltpu.VMEM((D,), jnp.float32),
            pltpu.VMEM((D,), jnp.float32),
            pltpu.SemaphoreType.DMA,
        ],
    )
    def _sc_combine(mem_hbm, idx_hbm, w_hbm, den_hbm, ps_hbm, out_hbm,
                    idx_v, rows_v, w_v, den_v, ps_v, out_v, sem):
        _sc_body(mem_hbm, idx_hbm, w_hbm, den_hbm, ps_hbm, out_hbm,
                 idx_v, rows_v, w_v, den_v, ps_v, out_v, sem)

    return _sc_combine


def _sc_body(mem_hbm, idx_hbm, w_hbm, den_hbm, ps_hbm, out_hbm,
             idx_v, rows_v, w_v, den_v, ps_v, out_v, sem):
    wid = lax.axis_index("s") * 2 + lax.axis_index("c")

    @pl.when(wid < W)
    def _():
        pltpu.sync_copy(idx_hbm.at[pl.ds(wid * 16, 16)], idx_v)
        pltpu.sync_copy(w_hbm.at[pl.ds(wid * 16, 16)], w_v)
        pltpu.sync_copy(den_hbm.at[pl.ds(wid * 16, 16)], den_v)
        pltpu.sync_copy(ps_hbm.at[pl.ds(wid * D, D)], ps_v)
        pltpu.async_copy(mem_hbm.at[idx_v], rows_v, sem).wait()

        wvec = w_v[...]
        dvec = den_v[...]
        wks = [
            wvec.at[jnp.full((16,), k, jnp.int32)].get(mode="promise_in_bounds")
            for k in range(K)
        ]
        for c in range(D // 16):
            acc = ps_v[pl.ds(c * 16, 16)]
            for k in range(K):
                acc = acc + wks[k] * rows_v[k, pl.ds(c * 16, 16)]
            out_v[pl.ds(c * 16, 16)] = acc / dvec
        pltpu.sync_copy(out_v, out_hbm.at[pl.ds(wid * D, D)])


def kernel(support, memory_encoded):
    sup = support.reshape(S * W, D)
    ps, w16, i16, den16 = _scores_stage(sup, memory_encoded)
    proto = _build_sc_combine()(
        memory_encoded,
        i16.reshape(-1),
        w16.reshape(-1),
        den16.reshape(-1),
        ps.reshape(-1),
    )
    return proto.reshape(1, W, D)


# speedup vs baseline: 1.0953x; 1.0040x over previous
import functools

import jax
import jax.numpy as jnp
from jax import lax
from jax.experimental import pallas as pl
from jax.experimental.pallas import tpu as pltpu
from jax.experimental.pallas import tpu_sc as plsc

D = 128
W = 5
S = 5
M = 65536
K = 8
BLK = 32768
G = M // BLK
NCOLS = M + 128


def _rsqrt_precise(x):
    y = lax.rsqrt(x)
    y = y * (1.5 - 0.5 * x * y * y)
    return y * (1.5 - 0.5 * x * y * y)


def _tc_body(sup_ref, mem_ref, ps_ref, w_ref, idx_ref, den_ref,
             tv_ref, ti_ref):
    i = pl.program_id(0)

    sup = sup_ref[...]
    n2s = jnp.sum(sup * sup, axis=1, keepdims=True)
    shat = sup * _rsqrt_precise(jnp.maximum(n2s, 1e-24))
    q = (shat[0:5] + shat[5:10] + shat[10:15] + shat[15:20] + shat[20:25]) * 0.2

    mblk = mem_ref[...]
    n2m = lax.dot_general(
        jnp.ones((1, D), jnp.float32), mblk * mblk,
        (((1,), (1,)), ((), ())),
        preferred_element_type=jnp.float32,
    )
    rn = _rsqrt_precise(jnp.maximum(n2m, 1e-24))
    sc = lax.dot_general(
        q, mblk, (((1,), (1,)), ((), ())),
        preferred_element_type=jnp.float32,
    ) * rn

    giota = lax.broadcasted_iota(jnp.int32, (W, BLK), 1) + i * BLK
    bvals, bidxs = [], []
    for _ in range(K):
        m = jnp.max(sc, axis=1, keepdims=True)
        am = jnp.min(jnp.where(sc >= m, giota, 2**30),
                     axis=1, keepdims=True)
        bvals.append(m)
        bidxs.append(am)
        sc = jnp.where(giota == am, -1e30, sc)
    bv = jnp.concatenate(bvals, axis=1)
    bi = jnp.concatenate(bidxs, axis=1)

    def _pad_store(v8, i8):
        tv_ref[...] = jnp.concatenate(
            [v8, jnp.full((W, 128 - K), -1e30, jnp.float32)], axis=1)
        ti_ref[...] = jnp.concatenate(
            [i8, jnp.zeros((W, 128 - K), jnp.int32)], axis=1)

    def _top8_of(cand_v, cand_i, n):
        liota = lax.broadcasted_iota(jnp.int32, (W, n), 1)
        vals, idxs = [], []
        for _ in range(K):
            m = jnp.max(cand_v, axis=1, keepdims=True)
            am = jnp.min(jnp.where(cand_v >= m, liota, 2**30),
                         axis=1, keepdims=True)
            gi = jnp.sum(jnp.where(liota == am, cand_i, 0),
                         axis=1, keepdims=True)
            vals.append(m)
            idxs.append(gi)
            cand_v = jnp.where(liota == am, -1e30, cand_v)
        return (jnp.concatenate(vals, axis=1),
                jnp.concatenate(idxs, axis=1))

    @pl.when(i == 0)
    def _init():
        _pad_store(bv, bi)

    @pl.when(i > 0)
    def _merge():
        rv = tv_ref[...][:, 0:K]
        ri = ti_ref[...][:, 0:K]
        mv, mi = _top8_of(jnp.concatenate([rv, bv], axis=1),
                          jnp.concatenate([ri, bi], axis=1), 2 * K)
        _pad_store(mv, mi)

    @pl.when(i == G - 1)
    def _finalize():
        cols = [
            jnp.sum(q * shat[j * W:(j + 1) * W], axis=1, keepdims=True)
            for j in range(S)
        ]
        supsc = jnp.concatenate(cols, axis=1)
        sup_gidx = lax.broadcasted_iota(jnp.int32, (W, S), 1) + M
        rv = tv_ref[...][:, 0:K]
        ri = ti_ref[...][:, 0:K]
        pad = jnp.full((W, 3), -1e30, jnp.float32)
        padi = jnp.zeros((W, 3), jnp.int32)
        V, I = _top8_of(jnp.concatenate([rv, supsc, pad], axis=1),
                        jnp.concatenate([ri, sup_gidx, padi], axis=1), 2 * K)

        is_mem = I < M
        w8 = jnp.where(is_mem, V, 0.0)
        i8 = jnp.where(is_mem, I, 0)
        zf = jnp.zeros((W, K), jnp.float32)
        zi = jnp.zeros((W, K), jnp.int32)
        w_ref[...] = jnp.concatenate([w8, zf], axis=1)
        idx_ref[...] = jnp.concatenate([i8, zi], axis=1)
        den = jnp.sum(V, axis=1, keepdims=True)
        den_ref[...] = jnp.broadcast_to(den, (W, 16))

        ps = jnp.zeros((W, D), jnp.float32)
        for j in range(S):
            swj = jnp.sum(jnp.where(I == M + j, V, 0.0), axis=1, keepdims=True)
            ps = ps + swj * sup[j * W:(j + 1) * W]
        ps_ref[...] = ps


def _scores_stage(sup, memory):
    return pl.pallas_call(
        _tc_body,
        grid=(G,),
        in_specs=[
            pl.BlockSpec((S * W, D), lambda i: (0, 0)),
            pl.BlockSpec((BLK, D), lambda i: (i, 0)),
        ],
        out_specs=[
            pl.BlockSpec((W, D), lambda i: (0, 0)),
            pl.BlockSpec((W, 16), lambda i: (0, 0)),
            pl.BlockSpec((W, 16), lambda i: (0, 0)),
            pl.BlockSpec((W, 16), lambda i: (0, 0)),
        ],
        out_shape=[
            jax.ShapeDtypeStruct((W, D), jnp.float32),
            jax.ShapeDtypeStruct((W, 16), jnp.float32),
            jax.ShapeDtypeStruct((W, 16), jnp.int32),
            jax.ShapeDtypeStruct((W, 16), jnp.float32),
        ],
        scratch_shapes=[
            pltpu.VMEM((W, 128), jnp.float32),
            pltpu.VMEM((W, 128), jnp.int32),
        ],
    )(sup, memory)


@functools.cache
def _build_sc_combine():
    @functools.partial(
        pl.kernel,
        mesh=plsc.VectorSubcoreMesh(core_axis_name="c", subcore_axis_name="s"),
        out_type=jax.ShapeDtypeStruct((W * D,), jnp.float32),
        scratch_types=[
            pltpu.VMEM((16,), jnp.int32),
            pltpu.VMEM((16, D), jnp.float32),
            pltpu.VMEM((16,), jnp.float32),
            pltpu.VMEM((16,), jnp.float32),
            pltpu.VMEM((D,), jnp.float32),
            pltpu.VMEM((D,), jnp.float32),
            pltpu.SemaphoreType.DMA,
        ],
    )
    def _sc_combine(mem_hbm, idx_hbm, w_hbm, den_hbm, ps_hbm, out_hbm,
                    idx_v, rows_v, w_v, den_v, ps_v, out_v, sem):
        _sc_body(mem_hbm, idx_hbm, w_hbm, den_hbm, ps_hbm, out_hbm,
                 idx_v, rows_v, w_v, den_v, ps_v, out_v, sem)

    return _sc_combine


def _sc_body(mem_hbm, idx_hbm, w_hbm, den_hbm, ps_hbm, out_hbm,
             idx_v, rows_v, w_v, den_v, ps_v, out_v, sem):
    wid = lax.axis_index("s") * 2 + lax.axis_index("c")

    @pl.when(wid < W)
    def _():
        pltpu.sync_copy(idx_hbm.at[pl.ds(wid * 16, 16)], idx_v)
        pltpu.sync_copy(w_hbm.at[pl.ds(wid * 16, 16)], w_v)
        pltpu.sync_copy(den_hbm.at[pl.ds(wid * 16, 16)], den_v)
        pltpu.sync_copy(ps_hbm.at[pl.ds(wid * D, D)], ps_v)
        pltpu.async_copy(mem_hbm.at[idx_v], rows_v, sem).wait()

        wvec = w_v[...]
        dvec = den_v[...]
        wks = [
            wvec.at[jnp.full((16,), k, jnp.int32)].get(mode="promise_in_bounds")
            for k in range(K)
        ]
        for c in range(D // 16):
            acc = ps_v[pl.ds(c * 16, 16)]
            for k in range(K):
                acc = acc + wks[k] * rows_v[k, pl.ds(c * 16, 16)]
            out_v[pl.ds(c * 16, 16)] = acc / dvec
        pltpu.sync_copy(out_v, out_hbm.at[pl.ds(wid * D, D)])


def kernel(support, memory_encoded):
    sup = support.reshape(S * W, D)
    ps, w16, i16, den16 = _scores_stage(sup, memory_encoded)
    proto = _build_sc_combine()(
        memory_encoded,
        i16.reshape(-1),
        w16.reshape(-1),
        den16.reshape(-1),
        ps.reshape(-1),
    )
    return proto.reshape(1, W, D)
